# Initial kernel scaffold; baseline (speedup 1.0000x reference)
import jax
import jax.numpy as jnp
from jax.experimental import pallas as pl


def kernel(x, embedding):
    raise NotImplementedError("write your pallas kernel here")


# speedup vs baseline: 11.8233x; 11.8233x over previous
import jax
import jax.numpy as jnp
from jax.experimental import pallas as pl
from jax.experimental.pallas import tpu as pltpu

BATCH = 4096
NUM_RINGS = 200
EMBED_DIM = 128
B_BLK = 64


def _add_body(x_ref, emb_ref, out_ref):
    out_ref[...] = x_ref[...] + emb_ref[...]


def kernel(x, embedding):
    grid = (BATCH // B_BLK,)
    return pl.pallas_call(
        _add_body,
        grid=grid,
        in_specs=[
            pl.BlockSpec((B_BLK, NUM_RINGS, EMBED_DIM), lambda i: (i, 0, 0)),
            pl.BlockSpec((1, NUM_RINGS, EMBED_DIM), lambda i: (0, 0, 0)),
        ],
        out_specs=pl.BlockSpec((B_BLK, NUM_RINGS, EMBED_DIM), lambda i: (i, 0, 0)),
        out_shape=jax.ShapeDtypeStruct((BATCH, NUM_RINGS, EMBED_DIM), x.dtype),
    )(x, embedding[None])
